# baseline (speedup 1.0000x reference)
import functools

import jax
import jax.numpy as jnp
from jax import lax
from jax.experimental import pallas as pl
from jax.experimental.pallas import tpu as pltpu
from jax.experimental.pallas import tpu_sc as plsc

_DELTA_V = 0.5
_DELTA_D = 3.0
_ALPHA = 1.0
_BETA = 1.0
_GAMA = 0.001

_B, _D, _N = 4, 32, 262144
_L = 4
_R = 5
_NC, _NS = 2, 16
_NW = _NC * _NS
_CHUNK = _N // _NW
_T = 2048
_NT = _CHUNK // _T
_VPT = _T // 16

_PB = _R * _D + _L * 16
_ROW = _B * _PB

_mesh = plsc.VectorSubcoreMesh(core_axis_name="c", subcore_axis_name="s")


def _zeros16():
    return jnp.zeros((16,), jnp.float32)


@functools.partial(
    pl.kernel,
    out_type=jax.ShapeDtypeStruct((_NW, _ROW), jnp.float32),
    mesh=_mesh,
    compiler_params=pltpu.CompilerParams(needs_layout_passes=False),
    scratch_types=[
        pltpu.VMEM((_D, _T), jnp.float32),
        pltpu.VMEM((_T,), jnp.int32),
        pltpu.VMEM((_R * _D,), jnp.float32),
        pltpu.VMEM((_ROW,), jnp.float32),
    ],
)
def _pass_a(pred_hbm, tgt_hbm, out_hbm, xbuf, segbuf, acc, rowbuf):
    wid = lax.axis_index("s") * _NC + lax.axis_index("c")
    base0 = wid * _CHUNK
    for b in range(_B):
        for v in range(_R * _D // 16):
            acc[pl.ds(v * 16, 16)] = _zeros16()

        def tile_body(t, cnt):
            base = base0 + t * _T
            pltpu.sync_copy(pred_hbm.at[b, :, pl.ds(base, _T)], xbuf)
            pltpu.sync_copy(tgt_hbm.at[b, pl.ds(base, _T)], segbuf)

            def body(j, cnt):
                c1, c2, c3, c4 = cnt
                off = j * 16
                seg = segbuf[pl.ds(off, 16)]
                one = jnp.ones((16,), jnp.float32)
                zero = _zeros16()
                c1 = c1 + jnp.where(seg == 1, one, zero)
                c2 = c2 + jnp.where(seg == 2, one, zero)
                c3 = c3 + jnp.where(seg == 3, one, zero)
                c4 = c4 + jnp.where(seg == 4, one, zero)
                idx0 = seg * _D
                for d in range(_D):
                    x = xbuf[d, pl.ds(off, 16)]
                    plsc.addupdate_scatter(acc, [idx0 + d], x)
                return (c1, c2, c3, c4)

            return lax.fori_loop(0, _VPT, body, cnt)

        z = _zeros16()
        cnt = lax.fori_loop(0, _NT, tile_body, (z, z, z, z))
        for v in range(_R * _D // 16):
            rowbuf[pl.ds(b * _PB + v * 16, 16)] = acc[pl.ds(v * 16, 16)]
        for l in range(_L):
            rowbuf[pl.ds(b * _PB + _R * _D + l * 16, 16)] = cnt[l]
    pltpu.sync_copy(rowbuf, out_hbm.at[wid])


@functools.partial(
    pl.kernel,
    out_type=(
        jax.ShapeDtypeStruct((_NW, _B * 16), jnp.float32),
        jax.ShapeDtypeStruct((_B * _R * _D,), jnp.float32),
        jax.ShapeDtypeStruct((_B * 16,), jnp.float32),
    ),
    mesh=_mesh,
    compiler_params=pltpu.CompilerParams(needs_layout_passes=False),
    scratch_types=[
        pltpu.VMEM((_D, _T), jnp.float32),
        pltpu.VMEM((_T,), jnp.int32),
        pltpu.VMEM((_NW, _ROW), jnp.float32),
        pltpu.VMEM((_ROW,), jnp.float32),
        pltpu.VMEM((_B * _R * _D,), jnp.float32),
        pltpu.VMEM((_B * 16,), jnp.float32),
        pltpu.VMEM((_B * 16,), jnp.float32),
    ],
)
def _pass_b(pred_hbm, tgt_hbm, part_hbm, var_out, cent_out, cnt_out,
            xbuf, segbuf, partbuf, red, cent, cntv, varv):
    wid = lax.axis_index("s") * _NC + lax.axis_index("c")
    base0 = wid * _CHUNK

    pltpu.sync_copy(part_hbm, partbuf)

    def red_body(v, _):
        off = v * 16
        s = partbuf[0, pl.ds(off, 16)]
        for w in range(1, _NW):
            s = s + partbuf[w, pl.ds(off, 16)]
        red[pl.ds(off, 16)] = s
        return 0

    lax.fori_loop(0, _ROW // 16, red_body, 0)

    lanes = lax.iota(jnp.int32, 16)
    for b in range(_B):
        for h in range(_D // 16):
            cent[pl.ds(b * _R * _D + h * 16, 16)] = _zeros16()
        cv = _zeros16()
        for l in range(1, _R):
            cnt_l = jnp.sum(red[pl.ds(b * _PB + _R * _D + (l - 1) * 16, 16)])
            cv_safe = jnp.maximum(jnp.full((16,), cnt_l), 1.0)
            inv = plsc.bitcast(
                jnp.int32(0x7EF311C3) - plsc.bitcast(cv_safe, jnp.int32),
                jnp.float32)
            for _ in range(3):
                inv = inv * (2.0 - cv_safe * inv)
            for h in range(_D // 16):
                svec = red[pl.ds(b * _PB + l * _D + h * 16, 16)]
                cent[pl.ds(b * _R * _D + l * _D + h * 16, 16)] = svec * inv
            cv = jnp.where(lanes == (l - 1), jnp.full((16,), cnt_l), cv)
        cntv[pl.ds(b * 16, 16)] = cv

    for b in range(_B):
        def tile_body(t, acc):
            base = base0 + t * _T
            pltpu.sync_copy(pred_hbm.at[b, :, pl.ds(base, _T)], xbuf)
            pltpu.sync_copy(tgt_hbm.at[b, pl.ds(base, _T)], segbuf)

            def body(j, acc):
                a1, a2, a3, a4 = acc
                off = j * 16
                seg = segbuf[pl.ds(off, 16)]
                gidx0 = seg * _D + (b * _R * _D)
                q = _zeros16()
                for d in range(_D):
                    x = xbuf[d, pl.ds(off, 16)]
                    c = plsc.load_gather(cent, [gidx0 + d])
                    diff = x - c
                    q = q + diff * diff
                qc = jnp.maximum(q, jnp.float32(1e-20))
                i = plsc.bitcast(qc, jnp.int32)
                y = plsc.bitcast(jnp.int32(0x5F3759DF) - (i >> 1), jnp.float32)
                h = qc * 0.5
                for _ in range(3):
                    y = y * (1.5 - h * y * y)
                nrm = qc * y
                r = jnp.maximum(nrm - _DELTA_V, 0.0)
                r2 = r * r
                zero = _zeros16()
                a1 = a1 + jnp.where(seg == 1, r2, zero)
                a2 = a2 + jnp.where(seg == 2, r2, zero)
                a3 = a3 + jnp.where(seg == 3, r2, zero)
                a4 = a4 + jnp.where(seg == 4, r2, zero)
                return (a1, a2, a3, a4)

            return lax.fori_loop(0, _VPT, body, acc)

        z = _zeros16()
        acc = lax.fori_loop(0, _NT, tile_body, (z, z, z, z))
        vv = _zeros16()
        for l in range(_L):
            s = jnp.sum(acc[l])
            vv = jnp.where(lanes == l, jnp.full((16,), s), vv)
        varv[pl.ds(b * 16, 16)] = vv

    pltpu.sync_copy(varv, var_out.at[wid])

    @pl.when(wid == 0)
    def _():
        pltpu.sync_copy(cent, cent_out)
        pltpu.sync_copy(cntv, cnt_out)


def _final_tc(cent_ref, cnt_ref, var_ref, out_ref):
    vsum = jnp.sum(var_ref[...], axis=0, keepdims=True)
    var_loss = jnp.float32(0.0)
    dist_loss = jnp.float32(0.0)
    reg_loss = jnp.float32(0.0)
    rows = lax.broadcasted_iota(jnp.int32, (_L, _L), 0)
    cols = lax.broadcasted_iota(jnp.int32, (_L, _L), 1)
    eye = rows == cols
    for b in range(_B):
        cb = cent_ref[b * _R + 1:b * _R + _R, :]
        cnt2 = cnt_ref[b:b + 1, 0:_L]
        present = cnt2 > 0
        pf = present.astype(jnp.float32)
        num_id = jnp.sum(pf)
        num_id_safe = jnp.where(num_id > 0, num_id, 1.0)
        cnt_safe = jnp.where(present, cnt2, 1.0)
        vb = vsum[:, b * 16:b * 16 + _L]
        var_loss = var_loss + jnp.sum(
            jnp.where(present, vb / cnt_safe / num_id_safe, 0.0))
        qn = jnp.sum(cb * cb, axis=1, keepdims=True)
        g = jnp.dot(cb, cb.T, preferred_element_type=jnp.float32)
        d2 = jnp.maximum(qn + qn.T - 2.0 * g, 0.0)
        dist = jnp.sqrt(jnp.where(eye, 1.0, d2))
        dist = jnp.where(eye, _DELTA_D, dist)
        pair = pf.T * pf
        hinge = jnp.maximum(_DELTA_D - dist, 0.0)
        pair_sum = jnp.sum(hinge * hinge * pair)
        denom = jnp.where(num_id > 1, num_id_safe * (num_id_safe - 1.0), 1.0)
        dist_loss = dist_loss + jnp.where(num_id > 1, pair_sum / denom / 2.0, 0.0)
        norms = jnp.sqrt(qn)
        reg_loss = reg_loss + jnp.where(num_id > 0, jnp.sum(norms) / num_id_safe, 0.0)
    total = (_ALPHA * var_loss + _BETA * dist_loss + _GAMA * reg_loss) / _B
    out_ref[:, :] = total[None, None]


def kernel(pred, target):
    tgt = target.astype(jnp.int32)
    part = _pass_a(pred, tgt)
    var_out, cent_out, cnt_out = _pass_b(pred, tgt, part)
    loss = pl.pallas_call(
        _final_tc,
        out_shape=jax.ShapeDtypeStruct((1, 1), jnp.float32),
    )(cent_out.reshape(_B * _R, _D), cnt_out.reshape(_B, 16), var_out)
    return loss[0, 0]
